# Initial kernel scaffold; baseline (speedup 1.0000x reference)
import jax
import jax.numpy as jnp
from jax.experimental import pallas as pl


def kernel(x, edge_index, edge_attr, enc_n_W0, enc_n_b0, enc_n_W1, enc_n_b1, enc_e_W0, enc_e_b0, enc_e_W1, enc_e_b1, em_W0, em_b0, em_W1, em_b1, fo_W0, fo_b0, fo_W1, fo_b1, fi_W0, fi_b0, fi_W1, fi_b1, nm_W0, nm_b0, cls_W0, cls_b0, cls_W1, cls_b1):
    raise NotImplementedError("write your pallas kernel here")


# speedup vs baseline: 2.9807x; 2.9807x over previous
import functools

import jax
import jax.numpy as jnp
from jax import lax
from jax.experimental import pallas as pl
from jax.experimental.pallas import tpu as pltpu
from jax.experimental.pallas import tpu_sc as plsc

N = 10000
N2 = 10240
E = 320000
ROW_W = 80
NWORK = 32
RPW = E // ROW_W // NWORK
TPC = 16
NPT = N2 // TPC

EB = 2560
NEB = E // EB

GCH = 25
GNC = RPW // GCH
SCH = 5
SNC = RPW // SCH


@functools.lru_cache(maxsize=None)
def _sc_kernels():
    mesh = plsc.VectorSubcoreMesh(core_axis_name="c", subcore_axis_name="s",
                                  num_cores=2, num_subcores=16)

    @functools.partial(
        pl.kernel,
        out_type=(jax.ShapeDtypeStruct((NWORK, RPW, ROW_W, 32), jnp.float32),
                  jax.ShapeDtypeStruct((NWORK, RPW, ROW_W, 32), jnp.float32)),
        mesh=mesh,
        compiler_params=pltpu.CompilerParams(use_tc_tiling_on_sc=False),
        scratch_types=[
            pltpu.VMEM((RPW, ROW_W), jnp.int32),
            pltpu.VMEM((GCH, ROW_W, 32), jnp.float32),
            pltpu.SemaphoreType.DMA,
        ],
    )
    def _gather_sc(hn_hbm, row_hbm, col_hbm, src_out, dst_out,
                   idx_v, rows_v, sem):
        wid = lax.axis_index("s") * 2 + lax.axis_index("c")
        for idx_hbm, out_hbm in ((row_hbm, src_out), (col_hbm, dst_out)):
            pltpu.sync_copy(idx_hbm.at[wid], idx_v)
            for i in range(GNC):
                cps = [pltpu.async_copy(hn_hbm.at[idx_v.at[i * GCH + j]],
                                        rows_v.at[j], sem)
                       for j in range(GCH)]
                for cp in cps:
                    cp.wait()
                pltpu.sync_copy(rows_v, out_hbm.at[wid, pl.ds(i * GCH, GCH)])

    @functools.partial(
        pl.kernel,
        out_type=jax.ShapeDtypeStruct((2, N2, 64), jnp.float32),
        mesh=mesh,
        compiler_params=pltpu.CompilerParams(use_tc_tiling_on_sc=False),
        scratch_types=[
            pltpu.VMEM((RPW, ROW_W), jnp.int32),
            pltpu.VMEM((SCH, ROW_W, 64), jnp.float32),
            pltpu.VMEM((NPT, 64), jnp.float32),
            pltpu.VMEM_SHARED((N2, 64), jnp.float32),
            pltpu.SemaphoreType.DMA,
        ],
    )
    def _scatter_sc(vals_hbm, row_hbm, zero_hbm, out_hbm,
                    idx_v, vals_v, zb_v, acc_sh, sem):
        c = lax.axis_index("c")
        s = lax.axis_index("s")
        wid = s * 2 + c
        pltpu.sync_copy(zero_hbm, zb_v)
        pltpu.sync_copy(zb_v, acc_sh.at[pl.ds(s * NPT, NPT)])
        plsc.subcore_barrier()
        pltpu.sync_copy(row_hbm.at[wid], idx_v)
        for i in range(SNC):
            pltpu.sync_copy(vals_hbm.at[wid, pl.ds(i * SCH, SCH)], vals_v)
            for j in range(SCH):
                pltpu.sync_copy(vals_v.at[j], acc_sh.at[idx_v.at[i * SCH + j]],
                                add=True)
        plsc.subcore_barrier()
        pltpu.sync_copy(acc_sh.at[pl.ds(s * NPT, NPT)],
                        out_hbm.at[c, pl.ds(s * NPT, NPT)])

    return _gather_sc, _scatter_sc


def _full(w):
    return pl.BlockSpec(w.shape, lambda i: (0,) * w.ndim)


def _relu(v):
    return jnp.maximum(v, 0.0)


def _edge_body(enc, last, *refs):
    it = iter(refs)
    src_ref, dst_ref, he_ref = next(it), next(it), next(it)
    if not last:
        row_ref, col_ref = next(it), next(it)
    if enc:
        eW0, eb0, eW1, eb1 = next(it), next(it), next(it), next(it)
    emW0, emb0, emW1, emb1 = next(it), next(it), next(it), next(it)
    if last:
        cW0, cb0, cW1, cb1 = next(it), next(it), next(it), next(it)
        lg_out = next(it)
    else:
        Wf0, bf0, Wf1, bf1 = next(it), next(it), next(it), next(it)
        he_out, ff_out = next(it), next(it)

    src, dst, he = src_ref[...], dst_ref[...], he_ref[...]
    if enc:
        he = _relu(he @ eW0[...] + eb0[...])
        he = _relu(he @ eW1[...] + eb1[...])
    t = jnp.concatenate([src, dst, he], axis=1)
    h1 = _relu(t @ emW0[...] + emb0[...])
    he2 = _relu(h1 @ emW1[...] + emb1[...])
    if last:
        c1 = _relu(he2 @ cW0[...] + cb0[...])
        lg_out[...] = c1 @ cW1[...] + cb1[...]
        return
    he_out[...] = he2
    f = jnp.concatenate([src, he2], axis=1)
    g = _relu(f @ Wf0[...] + bf0[...])
    ff = _relu(g @ Wf1[...] + bf1[...])
    r = row_ref[0]
    cc = col_ref[0]
    mi = (r > cc).astype(jnp.float32)
    mo = (r < cc).astype(jnp.float32)
    m = jnp.concatenate([jnp.broadcast_to(mi, (EB, 32)),
                         jnp.broadcast_to(mo, (EB, 32))], axis=1)
    ff_out[...] = ff * m


def _edge_call(enc, last, src, dst, he, row3, col3, weights):
    eb_specs = {32: pl.BlockSpec((EB, 32), lambda i: (i, 0)),
                16: pl.BlockSpec((EB, 16), lambda i: (i, 0))}
    idx_spec = pl.BlockSpec((1, EB, 1), lambda i: (i, 0, 0))
    in_specs = [eb_specs[32], eb_specs[32], eb_specs[16]]
    ops = [src, dst, he]
    if not last:
        in_specs += [idx_spec, idx_spec]
        ops += [row3, col3]
    in_specs += [_full(w) for w in weights]
    ops += list(weights)
    if last:
        out_shape = jax.ShapeDtypeStruct((E, 1), jnp.float32)
        out_specs = pl.BlockSpec((EB, 1), lambda i: (i, 0))
    else:
        out_shape = (jax.ShapeDtypeStruct((E, 16), jnp.float32),
                     jax.ShapeDtypeStruct((E, 64), jnp.float32))
        out_specs = (eb_specs[16], pl.BlockSpec((EB, 64), lambda i: (i, 0)))
    return pl.pallas_call(
        functools.partial(_edge_body, enc, last),
        grid=(NEB,),
        in_specs=in_specs,
        out_specs=out_specs,
        out_shape=out_shape,
    )(*ops)


def _encn_body(x_ref, W0, b0, W1, b1, out_ref):
    h = _relu(x_ref[...] @ W0[...] + b0[...])
    out_ref[...] = _relu(h @ W1[...] + b1[...])


def _node_body(p_ref, W0, b0, out_ref):
    a = p_ref[0] + p_ref[1]
    out_ref[...] = _relu(a @ W0[...] + b0[...])


def kernel(x, edge_index, edge_attr,
           enc_n_W0, enc_n_b0, enc_n_W1, enc_n_b1,
           enc_e_W0, enc_e_b0, enc_e_W1, enc_e_b1,
           em_W0, em_b0, em_W1, em_b1,
           fo_W0, fo_b0, fo_W1, fo_b1,
           fi_W0, fi_b0, fi_W1, fi_b1,
           nm_W0, nm_b0,
           cls_W0, cls_b0, cls_W1, cls_b1):
    f32 = jnp.float32
    row = edge_index[0]
    col = edge_index[1]
    row_r = row.reshape(NWORK, RPW, ROW_W)
    col_r = col.reshape(NWORK, RPW, ROW_W)
    row3 = row.reshape(NEB, EB, 1)
    col3 = col.reshape(NEB, EB, 1)
    zero = jnp.zeros((NPT, 64), f32)

    r1 = lambda b: b.reshape(1, -1)
    Wf0 = jnp.concatenate([fi_W0, fo_W0], axis=1)
    bf0 = r1(jnp.concatenate([fi_b0, fo_b0]))
    z = jnp.zeros((56, 32), f32)
    Wf1 = jnp.concatenate(
        [jnp.concatenate([fi_W1, z], axis=1),
         jnp.concatenate([z, fo_W1], axis=1)], axis=0)
    bf1 = r1(jnp.concatenate([fi_b1, fo_b1]))

    h_n = pl.pallas_call(
        _encn_body,
        out_shape=jax.ShapeDtypeStruct((N, 32), f32),
    )(x, enc_n_W0, r1(enc_n_b0), enc_n_W1, r1(enc_n_b1))
    h_n = jnp.concatenate([h_n, jnp.zeros((N2 - N, 32), f32)], axis=0)

    gather_sc, scatter_sc = _sc_kernels()
    mid_w = (em_W0, r1(em_b0), em_W1, r1(em_b1), Wf0, bf0, Wf1, bf1)
    he = edge_attr
    for step in range(3):
        src, dst = gather_sc(h_n, row_r, col_r)
        src = src.reshape(E, 32)
        dst = dst.reshape(E, 32)
        if step == 0:
            weights = (enc_e_W0, r1(enc_e_b0), enc_e_W1, r1(enc_e_b1)) + mid_w
        else:
            weights = mid_w
        he, ff = _edge_call(step == 0, False, src, dst, he, row3, col3, weights)
        partials = scatter_sc(ff.reshape(NWORK, RPW, ROW_W, 64), row_r, zero)
        h_n = pl.pallas_call(
            _node_body,
            out_shape=jax.ShapeDtypeStruct((N2, 32), f32),
        )(partials, nm_W0, r1(nm_b0))

    src, dst = gather_sc(h_n, row_r, col_r)
    src = src.reshape(E, 32)
    dst = dst.reshape(E, 32)
    weights = (em_W0, r1(em_b0), em_W1, r1(em_b1),
               cls_W0, r1(cls_b0), cls_W1, r1(cls_b1))
    return _edge_call(False, True, src, dst, he, None, None, weights)


# speedup vs baseline: 5.0423x; 1.6916x over previous
import functools

import jax
import jax.numpy as jnp
from jax import lax
from jax.experimental import pallas as pl
from jax.experimental.pallas import tpu as pltpu
from jax.experimental.pallas import tpu_sc as plsc

N = 10000
N2 = 10240
E = 320000
ROW_W = 80
NWORK = 32
RPW = E // ROW_W // NWORK
TPC = 16
NPT = N2 // TPC

EB = 2560
B2 = EB // 2
NEB = E // EB
EP2 = E // 2

GCH = 5
GNC = RPW // GCH
SCH = 5
SNC = RPW // SCH


@functools.lru_cache(maxsize=None)
def _sc_kernels():
    mesh = plsc.VectorSubcoreMesh(core_axis_name="c", subcore_axis_name="s",
                                  num_cores=2, num_subcores=16)

    @functools.partial(
        pl.kernel,
        out_type=(jax.ShapeDtypeStruct((NWORK, RPW, ROW_W, 64), jnp.float32),
                  jax.ShapeDtypeStruct((NWORK, RPW, ROW_W, 64), jnp.float32)),
        mesh=mesh,
        compiler_params=pltpu.CompilerParams(use_tc_tiling_on_sc=False),
        scratch_types=[
            pltpu.VMEM((RPW, ROW_W), jnp.int32),
            pltpu.VMEM((GCH, ROW_W, 64), jnp.float32),
            pltpu.SemaphoreType.DMA,
        ],
    )
    def _gather_sc(hn_hbm, ei_hbm, src_out, dst_out, idx_v, rows_v, sem):
        wid = lax.axis_index("s") * 2 + lax.axis_index("c")
        for which, out_hbm in ((0, src_out), (1, dst_out)):
            pltpu.sync_copy(ei_hbm.at[which, wid], idx_v)
            for i in range(GNC):
                cps = [pltpu.async_copy(hn_hbm.at[idx_v.at[i * GCH + j]],
                                        rows_v.at[j], sem)
                       for j in range(GCH)]
                for cp in cps:
                    cp.wait()
                pltpu.sync_copy(rows_v, out_hbm.at[wid, pl.ds(i * GCH, GCH)])

    @functools.partial(
        pl.kernel,
        out_type=jax.ShapeDtypeStruct((2, N2, 64), jnp.float32),
        mesh=mesh,
        compiler_params=pltpu.CompilerParams(use_tc_tiling_on_sc=False),
        scratch_types=[
            pltpu.VMEM((RPW, ROW_W), jnp.int32),
            pltpu.VMEM((SCH, ROW_W, 64), jnp.float32),
            pltpu.VMEM((NPT, 64), jnp.float32),
            pltpu.VMEM_SHARED((N2, 64), jnp.float32),
            pltpu.SemaphoreType.DMA,
        ],
    )
    def _scatter_sc(vals_hbm, ei_hbm, zero_hbm, out_hbm,
                    idx_v, vals_v, zb_v, acc_sh, sem):
        c = lax.axis_index("c")
        s = lax.axis_index("s")
        wid = s * 2 + c
        pltpu.sync_copy(zero_hbm, zb_v)
        pltpu.sync_copy(zb_v, acc_sh.at[pl.ds(s * NPT, NPT)])
        plsc.subcore_barrier()
        pltpu.sync_copy(ei_hbm.at[0, wid], idx_v)
        for i in range(SNC):
            pltpu.sync_copy(vals_hbm.at[wid, pl.ds(i * SCH, SCH)], vals_v)
            for j in range(SCH):
                pltpu.sync_copy(vals_v.at[j], acc_sh.at[idx_v.at[i * SCH + j]],
                                add=True)
        plsc.subcore_barrier()
        pltpu.sync_copy(acc_sh.at[pl.ds(s * NPT, NPT)],
                        out_hbm.at[c, pl.ds(s * NPT, NPT)])

    return _gather_sc, _scatter_sc


def _full(w):
    return pl.BlockSpec(w.shape, lambda i: (0,) * w.ndim)


def _relu(v):
    return jnp.maximum(v, 0.0)


def _edge_body(enc, last, *refs):
    it = iter(refs)
    src_ref, dst_ref, he_ref = next(it), next(it), next(it)
    if enc:
        eW0, eb0, eW1, eb1 = next(it), next(it), next(it), next(it)
    emS, emD, emE, emb0, emW1, emb1 = (next(it) for _ in range(6))
    if last:
        cW0, cb0, cW1, cb1 = next(it), next(it), next(it), next(it)
        lg_out = next(it)
    else:
        fS, fE, fb0, fW1, fb1 = (next(it) for _ in range(5))
        he_out, ff_out = next(it), next(it)

    src, dst, he = src_ref[...], dst_ref[...], he_ref[...]
    if enc:
        he = _relu(he @ eW0[...] + eb0[...])
        he = _relu(he @ eW1[...] + eb1[...])
    h1 = _relu(src @ emS[...] + dst @ emD[...] + he @ emE[...] + emb0[...])
    he2 = _relu(h1 @ emW1[...] + emb1[...])
    if last:
        c1 = _relu(he2 @ cW0[...] + cb0[...])
        lg_out[...] = c1 @ cW1[...] + cb1[...]
        return
    he_out[...] = he2
    g = _relu(src @ fS[...] + he2 @ fE[...] + fb0[...])
    ff = _relu(g @ fW1[...] + fb1[...])
    f32 = jnp.float32
    s0, s1 = src[:, 32:33], src[:, 96:97]
    d0, d1 = dst[:, 32:33], dst[:, 96:97]
    m = jnp.concatenate(
        [jnp.broadcast_to((s0 > d0).astype(f32), (B2, 32)),
         jnp.broadcast_to((s0 < d0).astype(f32), (B2, 32)),
         jnp.broadcast_to((s1 > d1).astype(f32), (B2, 32)),
         jnp.broadcast_to((s1 < d1).astype(f32), (B2, 32))], axis=1)
    ff_out[...] = ff * m


def _edge_call(enc, last, src, dst, he, weights):
    spec128 = pl.BlockSpec((B2, 128), lambda i: (i, 0))
    spec32 = pl.BlockSpec((B2, 32), lambda i: (i, 0))
    in_specs = [spec128, spec128, spec32]
    ops = [src, dst, he]
    in_specs += [_full(w) for w in weights]
    ops += list(weights)
    if last:
        out_shape = jax.ShapeDtypeStruct((EP2, 2), jnp.float32)
        out_specs = pl.BlockSpec((B2, 2), lambda i: (i, 0))
    else:
        out_shape = (jax.ShapeDtypeStruct((EP2, 32), jnp.float32),
                     jax.ShapeDtypeStruct((EP2, 128), jnp.float32))
        out_specs = (spec32, spec128)
    return pl.pallas_call(
        functools.partial(_edge_body, enc, last),
        grid=(NEB,),
        in_specs=in_specs,
        out_specs=out_specs,
        out_shape=out_shape,
    )(*ops)


def _encn_body(x_ref, W0, b0, W1, b1, out_ref):
    h = _relu(x_ref[...] @ W0[...] + b0[...])
    h = _relu(h @ W1[...] + b1[...])
    ids = lax.broadcasted_iota(jnp.int32, (N2, 1), 0).astype(jnp.float32)
    out_ref[...] = jnp.concatenate(
        [h, ids, jnp.zeros((N2, 31), jnp.float32)], axis=1)


def _node_body(p_ref, W0, b0, out_ref):
    a = p_ref[0] + p_ref[1]
    h = _relu(a @ W0[...] + b0[...])
    ids = lax.broadcasted_iota(jnp.int32, (N2, 1), 0).astype(jnp.float32)
    out_ref[...] = jnp.concatenate(
        [h, ids, jnp.zeros((N2, 31), jnp.float32)], axis=1)


def kernel(x, edge_index, edge_attr,
           enc_n_W0, enc_n_b0, enc_n_W1, enc_n_b1,
           enc_e_W0, enc_e_b0, enc_e_W1, enc_e_b1,
           em_W0, em_b0, em_W1, em_b1,
           fo_W0, fo_b0, fo_W1, fo_b1,
           fi_W0, fi_b0, fi_W1, fi_b1,
           nm_W0, nm_b0,
           cls_W0, cls_b0, cls_W1, cls_b1):
    f32 = jnp.float32
    ei = edge_index.reshape(2, NWORK, RPW, ROW_W)
    zero = jnp.zeros((NPT, 64), f32)

    def bd2(a):
        z = jnp.zeros_like(a)
        return jnp.concatenate(
            [jnp.concatenate([a, z], axis=1),
             jnp.concatenate([z, a], axis=1)], axis=0)

    def r2(b):
        return jnp.concatenate([b, b]).reshape(1, -1)

    def pad64(a):
        return jnp.concatenate([a, jnp.zeros_like(a)], axis=0)

    emS = bd2(pad64(em_W0[0:32]))
    emD = bd2(pad64(em_W0[32:64]))
    emE = bd2(em_W0[64:80])
    emW1 = bd2(em_W1)
    Wf0 = jnp.concatenate([fi_W0, fo_W0], axis=1)
    z56 = jnp.zeros((56, 32), f32)
    Wf1 = jnp.concatenate(
        [jnp.concatenate([fi_W1, z56], axis=1),
         jnp.concatenate([z56, fo_W1], axis=1)], axis=0)
    fS = bd2(pad64(Wf0[0:32]))
    fE = bd2(Wf0[32:48])
    fW1 = bd2(Wf1)
    fb0 = r2(jnp.concatenate([fi_b0, fo_b0]))
    fb1 = r2(jnp.concatenate([fi_b1, fo_b1]))

    r1 = lambda b: b.reshape(1, -1)
    x2 = jnp.concatenate([x, jnp.zeros((N2 - N, 128), f32)], axis=0)
    h_n = pl.pallas_call(
        _encn_body,
        out_shape=jax.ShapeDtypeStruct((N2, 64), f32),
    )(x2, enc_n_W0, r1(enc_n_b0), enc_n_W1, r1(enc_n_b1))

    gather_sc, scatter_sc = _sc_kernels()
    mid_w = (emS, emD, emE, r2(em_b0), emW1, r2(em_b1),
             fS, fE, fb0, fW1, fb1)
    he = edge_attr.reshape(EP2, 32)
    for step in range(3):
        src, dst = gather_sc(h_n, ei)
        src = src.reshape(EP2, 128)
        dst = dst.reshape(EP2, 128)
        if step == 0:
            weights = (bd2(enc_e_W0), r2(enc_e_b0),
                       bd2(enc_e_W1), r2(enc_e_b1)) + mid_w
        else:
            weights = mid_w
        he, ff = _edge_call(step == 0, False, src, dst, he, weights)
        partials = scatter_sc(ff.reshape(NWORK, RPW, ROW_W, 64), ei, zero)
        h_n = pl.pallas_call(
            _node_body,
            out_shape=jax.ShapeDtypeStruct((N2, 64), f32),
        )(partials, nm_W0, r1(nm_b0))

    src, dst = gather_sc(h_n, ei)
    src = src.reshape(EP2, 128)
    dst = dst.reshape(EP2, 128)
    weights = (emS, emD, emE, r2(em_b0), emW1, r2(em_b1),
               bd2(cls_W0), r2(cls_b0), bd2(cls_W1), r2(cls_b1))
    lg = _edge_call(False, True, src, dst, he, weights)
    return lg.reshape(E, 1)


# speedup vs baseline: 5.3586x; 1.0627x over previous
import functools

import jax
import jax.numpy as jnp
from jax import lax
from jax.experimental import pallas as pl
from jax.experimental.pallas import tpu as pltpu
from jax.experimental.pallas import tpu_sc as plsc

N = 10000
N2 = 10240
E = 320000
ROW_W = 80
NWORK = 32
RPW = E // ROW_W // NWORK
TPC = 16
NPT = N2 // TPC

EB = 6400
B2 = EB // 2
NEB = E // EB
EP2 = E // 2

GCH = 5
GNC = RPW // GCH
SCH = 5
SNC = RPW // SCH


@functools.lru_cache(maxsize=None)
def _sc_kernels():
    mesh = plsc.VectorSubcoreMesh(core_axis_name="c", subcore_axis_name="s",
                                  num_cores=2, num_subcores=16)

    @functools.partial(
        pl.kernel,
        out_type=(jax.ShapeDtypeStruct((NWORK, RPW, ROW_W, 64), jnp.float32),
                  jax.ShapeDtypeStruct((NWORK, RPW, ROW_W, 64), jnp.float32)),
        mesh=mesh,
        compiler_params=pltpu.CompilerParams(use_tc_tiling_on_sc=False),
        scratch_types=[
            pltpu.VMEM((RPW, ROW_W), jnp.int32),
            pltpu.VMEM((GCH, ROW_W, 64), jnp.float32),
            pltpu.SemaphoreType.DMA,
        ],
    )
    def _gather_sc(hn_hbm, ei_hbm, src_out, dst_out, idx_v, rows_v, sem):
        wid = lax.axis_index("s") * 2 + lax.axis_index("c")
        for which, out_hbm in ((0, src_out), (1, dst_out)):
            pltpu.sync_copy(ei_hbm.at[which, wid], idx_v)
            for i in range(GNC):
                cps = [pltpu.async_copy(hn_hbm.at[idx_v.at[i * GCH + j]],
                                        rows_v.at[j], sem)
                       for j in range(GCH)]
                for cp in cps:
                    cp.wait()
                pltpu.sync_copy(rows_v, out_hbm.at[wid, pl.ds(i * GCH, GCH)])

    @functools.partial(
        pl.kernel,
        out_type=jax.ShapeDtypeStruct((2, N2, 64), jnp.float32),
        mesh=mesh,
        compiler_params=pltpu.CompilerParams(use_tc_tiling_on_sc=False),
        scratch_types=[
            pltpu.VMEM((RPW, ROW_W), jnp.int32),
            pltpu.VMEM((SCH, ROW_W, 64), jnp.float32),
            pltpu.VMEM((NPT, 64), jnp.float32),
            pltpu.VMEM_SHARED((N2, 64), jnp.float32),
            pltpu.SemaphoreType.DMA,
        ],
    )
    def _scatter_sc(vals_hbm, ei_hbm, zero_hbm, out_hbm,
                    idx_v, vals_v, zb_v, acc_sh, sem):
        c = lax.axis_index("c")
        s = lax.axis_index("s")
        wid = s * 2 + c
        pltpu.sync_copy(zero_hbm, zb_v)
        pltpu.sync_copy(zb_v, acc_sh.at[pl.ds(s * NPT, NPT)])
        plsc.subcore_barrier()
        pltpu.sync_copy(ei_hbm.at[0, wid], idx_v)
        for i in range(SNC):
            pltpu.sync_copy(vals_hbm.at[wid, pl.ds(i * SCH, SCH)], vals_v)
            for j in range(SCH):
                pltpu.sync_copy(vals_v.at[j], acc_sh.at[idx_v.at[i * SCH + j]],
                                add=True)
        plsc.subcore_barrier()
        pltpu.sync_copy(acc_sh.at[pl.ds(s * NPT, NPT)],
                        out_hbm.at[c, pl.ds(s * NPT, NPT)])

    return _gather_sc, _scatter_sc


def _full(w):
    return pl.BlockSpec(w.shape, lambda i: (0,) * w.ndim)


def _relu(v):
    return jnp.maximum(v, 0.0)


def _edge_body(enc, last, *refs):
    it = iter(refs)
    src_ref, dst_ref, he_ref = next(it), next(it), next(it)
    if enc:
        eW0, eb0, eW1, eb1 = next(it), next(it), next(it), next(it)
    emS, emD, emE, emb0, emW1, emb1 = (next(it) for _ in range(6))
    if last:
        cW0, cb0, cW1, cb1 = next(it), next(it), next(it), next(it)
        lg_out = next(it)
    else:
        fS, fE, fb0, fW1, fb1 = (next(it) for _ in range(5))
        he_out, ff_out = next(it), next(it)

    src, dst, he = src_ref[...], dst_ref[...], he_ref[...]
    if enc:
        he = _relu(he @ eW0[...] + eb0[...])
        he = _relu(he @ eW1[...] + eb1[...])
    h1 = _relu(src @ emS[...] + dst @ emD[...] + he @ emE[...] + emb0[...])
    he2 = _relu(h1 @ emW1[...] + emb1[...])
    if last:
        c1 = _relu(he2 @ cW0[...] + cb0[...])
        lg_out[...] = c1 @ cW1[...] + cb1[...]
        return
    he_out[...] = he2
    g = _relu(src @ fS[...] + he2 @ fE[...] + fb0[...])
    ff = _relu(g @ fW1[...] + fb1[...])
    f32 = jnp.float32
    s0, s1 = src[:, 32:33], src[:, 96:97]
    d0, d1 = dst[:, 32:33], dst[:, 96:97]
    m = jnp.concatenate(
        [jnp.broadcast_to((s0 > d0).astype(f32), (B2, 32)),
         jnp.broadcast_to((s0 < d0).astype(f32), (B2, 32)),
         jnp.broadcast_to((s1 > d1).astype(f32), (B2, 32)),
         jnp.broadcast_to((s1 < d1).astype(f32), (B2, 32))], axis=1)
    ff_out[...] = ff * m


def _edge_call(enc, last, src, dst, he, weights):
    spec128 = pl.BlockSpec((B2, 128), lambda i: (i, 0))
    spec32 = pl.BlockSpec((B2, 32), lambda i: (i, 0))
    in_specs = [spec128, spec128, spec32]
    ops = [src, dst, he]
    in_specs += [_full(w) for w in weights]
    ops += list(weights)
    if last:
        out_shape = jax.ShapeDtypeStruct((EP2, 2), jnp.float32)
        out_specs = pl.BlockSpec((B2, 2), lambda i: (i, 0))
    else:
        out_shape = (jax.ShapeDtypeStruct((EP2, 32), jnp.float32),
                     jax.ShapeDtypeStruct((EP2, 128), jnp.float32))
        out_specs = (spec32, spec128)
    return pl.pallas_call(
        functools.partial(_edge_body, enc, last),
        grid=(NEB,),
        in_specs=in_specs,
        out_specs=out_specs,
        out_shape=out_shape,
    )(*ops)


def _encn_body(x_ref, W0, b0, W1, b1, out_ref):
    h = _relu(x_ref[...] @ W0[...] + b0[...])
    h = _relu(h @ W1[...] + b1[...])
    ids = lax.broadcasted_iota(jnp.int32, (N2, 1), 0).astype(jnp.float32)
    out_ref[...] = jnp.concatenate(
        [h, ids, jnp.zeros((N2, 31), jnp.float32)], axis=1)


def _node_body(p_ref, W0, b0, out_ref):
    a = p_ref[0] + p_ref[1]
    h = _relu(a @ W0[...] + b0[...])
    ids = lax.broadcasted_iota(jnp.int32, (N2, 1), 0).astype(jnp.float32)
    out_ref[...] = jnp.concatenate(
        [h, ids, jnp.zeros((N2, 31), jnp.float32)], axis=1)


def kernel(x, edge_index, edge_attr,
           enc_n_W0, enc_n_b0, enc_n_W1, enc_n_b1,
           enc_e_W0, enc_e_b0, enc_e_W1, enc_e_b1,
           em_W0, em_b0, em_W1, em_b1,
           fo_W0, fo_b0, fo_W1, fo_b1,
           fi_W0, fi_b0, fi_W1, fi_b1,
           nm_W0, nm_b0,
           cls_W0, cls_b0, cls_W1, cls_b1):
    f32 = jnp.float32
    ei = edge_index.reshape(2, NWORK, RPW, ROW_W)
    zero = jnp.zeros((NPT, 64), f32)

    def bd2(a):
        z = jnp.zeros_like(a)
        return jnp.concatenate(
            [jnp.concatenate([a, z], axis=1),
             jnp.concatenate([z, a], axis=1)], axis=0)

    def r2(b):
        return jnp.concatenate([b, b]).reshape(1, -1)

    def pad64(a):
        return jnp.concatenate([a, jnp.zeros_like(a)], axis=0)

    emS = bd2(pad64(em_W0[0:32]))
    emD = bd2(pad64(em_W0[32:64]))
    emE = bd2(em_W0[64:80])
    emW1 = bd2(em_W1)
    Wf0 = jnp.concatenate([fi_W0, fo_W0], axis=1)
    z56 = jnp.zeros((56, 32), f32)
    Wf1 = jnp.concatenate(
        [jnp.concatenate([fi_W1, z56], axis=1),
         jnp.concatenate([z56, fo_W1], axis=1)], axis=0)
    fS = bd2(pad64(Wf0[0:32]))
    fE = bd2(Wf0[32:48])
    fW1 = bd2(Wf1)
    fb0 = r2(jnp.concatenate([fi_b0, fo_b0]))
    fb1 = r2(jnp.concatenate([fi_b1, fo_b1]))

    r1 = lambda b: b.reshape(1, -1)
    x2 = jnp.concatenate([x, jnp.zeros((N2 - N, 128), f32)], axis=0)
    h_n = pl.pallas_call(
        _encn_body,
        out_shape=jax.ShapeDtypeStruct((N2, 64), f32),
    )(x2, enc_n_W0, r1(enc_n_b0), enc_n_W1, r1(enc_n_b1))

    gather_sc, scatter_sc = _sc_kernels()
    mid_w = (emS, emD, emE, r2(em_b0), emW1, r2(em_b1),
             fS, fE, fb0, fW1, fb1)
    he = edge_attr.reshape(EP2, 32)
    for step in range(3):
        src, dst = gather_sc(h_n, ei)
        src = src.reshape(EP2, 128)
        dst = dst.reshape(EP2, 128)
        if step == 0:
            weights = (bd2(enc_e_W0), r2(enc_e_b0),
                       bd2(enc_e_W1), r2(enc_e_b1)) + mid_w
        else:
            weights = mid_w
        he, ff = _edge_call(step == 0, False, src, dst, he, weights)
        partials = scatter_sc(ff.reshape(NWORK, RPW, ROW_W, 64), ei, zero)
        h_n = pl.pallas_call(
            _node_body,
            out_shape=jax.ShapeDtypeStruct((N2, 64), f32),
        )(partials, nm_W0, r1(nm_b0))

    src, dst = gather_sc(h_n, ei)
    src = src.reshape(EP2, 128)
    dst = dst.reshape(EP2, 128)
    weights = (emS, emD, emE, r2(em_b0), emW1, r2(em_b1),
               bd2(cls_W0), r2(cls_b0), bd2(cls_W1), r2(cls_b1))
    lg = _edge_call(False, True, src, dst, he, weights)
    return lg.reshape(E, 1)


# speedup vs baseline: 5.4443x; 1.0160x over previous
import functools

import jax
import jax.numpy as jnp
from jax import lax
from jax.experimental import pallas as pl
from jax.experimental.pallas import tpu as pltpu
from jax.experimental.pallas import tpu_sc as plsc

N = 10000
N2 = 10240
E = 320000
ROW_W = 80
NWORK = 32
RPW = E // ROW_W // NWORK
TPC = 16
NPT = N2 // TPC

EB = 12800
B2 = EB // 2
NEB = E // EB
EP2 = E // 2

GCH = 5
GNC = RPW // GCH
SCH = 5
SNC = RPW // SCH


@functools.lru_cache(maxsize=None)
def _sc_kernels():
    mesh = plsc.VectorSubcoreMesh(core_axis_name="c", subcore_axis_name="s",
                                  num_cores=2, num_subcores=16)

    @functools.partial(
        pl.kernel,
        out_type=(jax.ShapeDtypeStruct((NWORK, RPW, ROW_W, 64), jnp.float32),
                  jax.ShapeDtypeStruct((NWORK, RPW, ROW_W, 64), jnp.float32)),
        mesh=mesh,
        compiler_params=pltpu.CompilerParams(use_tc_tiling_on_sc=False),
        scratch_types=[
            pltpu.VMEM((RPW, ROW_W), jnp.int32),
            pltpu.VMEM((GCH, ROW_W, 64), jnp.float32),
            pltpu.SemaphoreType.DMA,
        ],
    )
    def _gather_sc(hn_hbm, ei_hbm, src_out, dst_out, idx_v, rows_v, sem):
        wid = lax.axis_index("s") * 2 + lax.axis_index("c")
        for which, out_hbm in ((0, src_out), (1, dst_out)):
            pltpu.sync_copy(ei_hbm.at[which, wid], idx_v)
            for i in range(GNC):
                cps = [pltpu.async_copy(hn_hbm.at[idx_v.at[i * GCH + j]],
                                        rows_v.at[j], sem)
                       for j in range(GCH)]
                for cp in cps:
                    cp.wait()
                pltpu.sync_copy(rows_v, out_hbm.at[wid, pl.ds(i * GCH, GCH)])

    @functools.partial(
        pl.kernel,
        out_type=jax.ShapeDtypeStruct((2, N2, 64), jnp.float32),
        mesh=mesh,
        compiler_params=pltpu.CompilerParams(use_tc_tiling_on_sc=False),
        scratch_types=[
            pltpu.VMEM((RPW, ROW_W), jnp.int32),
            pltpu.VMEM((SCH, ROW_W, 64), jnp.float32),
            pltpu.VMEM((NPT, 64), jnp.float32),
            pltpu.VMEM_SHARED((N2, 64), jnp.float32),
            pltpu.SemaphoreType.DMA,
        ],
    )
    def _scatter_sc(vals_hbm, ei_hbm, zero_hbm, out_hbm,
                    idx_v, vals_v, zb_v, acc_sh, sem):
        c = lax.axis_index("c")
        s = lax.axis_index("s")
        wid = s * 2 + c
        pltpu.sync_copy(zero_hbm, zb_v)
        pltpu.sync_copy(zb_v, acc_sh.at[pl.ds(s * NPT, NPT)])
        plsc.subcore_barrier()
        pltpu.sync_copy(ei_hbm.at[0, wid], idx_v)
        for i in range(SNC):
            pltpu.sync_copy(vals_hbm.at[wid, pl.ds(i * SCH, SCH)], vals_v)
            for j in range(SCH):
                pltpu.sync_copy(vals_v.at[j], acc_sh.at[idx_v.at[i * SCH + j]],
                                add=True)
        plsc.subcore_barrier()
        pltpu.sync_copy(acc_sh.at[pl.ds(s * NPT, NPT)],
                        out_hbm.at[c, pl.ds(s * NPT, NPT)])

    return _gather_sc, _scatter_sc


def _full(w):
    return pl.BlockSpec(w.shape, lambda i: (0,) * w.ndim)


def _relu(v):
    return jnp.maximum(v, 0.0)


def _edge_body(enc, last, *refs):
    it = iter(refs)
    src_ref, dst_ref, he_ref = next(it), next(it), next(it)
    if enc:
        eW0, eb0, eW1, eb1 = next(it), next(it), next(it), next(it)
    emS, emD, emE, emb0, emW1, emb1 = (next(it) for _ in range(6))
    if last:
        cW0, cb0, cW1, cb1 = next(it), next(it), next(it), next(it)
        lg_out = next(it)
    else:
        fS, fE, fb0, fW1, fb1 = (next(it) for _ in range(5))
        he_out, ff_out = next(it), next(it)

    src, dst, he = src_ref[...], dst_ref[...], he_ref[...]
    if enc:
        he = _relu(he @ eW0[...] + eb0[...])
        he = _relu(he @ eW1[...] + eb1[...])
    h1 = _relu(src @ emS[...] + dst @ emD[...] + he @ emE[...] + emb0[...])
    he2 = _relu(h1 @ emW1[...] + emb1[...])
    if last:
        c1 = _relu(he2 @ cW0[...] + cb0[...])
        lg_out[...] = c1 @ cW1[...] + cb1[...]
        return
    he_out[...] = he2
    g = _relu(src @ fS[...] + he2 @ fE[...] + fb0[...])
    ff = _relu(g @ fW1[...] + fb1[...])
    f32 = jnp.float32
    s0, s1 = src[:, 32:33], src[:, 96:97]
    d0, d1 = dst[:, 32:33], dst[:, 96:97]
    m = jnp.concatenate(
        [jnp.broadcast_to((s0 > d0).astype(f32), (B2, 32)),
         jnp.broadcast_to((s0 < d0).astype(f32), (B2, 32)),
         jnp.broadcast_to((s1 > d1).astype(f32), (B2, 32)),
         jnp.broadcast_to((s1 < d1).astype(f32), (B2, 32))], axis=1)
    ff_out[...] = ff * m


def _edge_call(enc, last, src, dst, he, weights):
    spec128 = pl.BlockSpec((B2, 128), lambda i: (i, 0))
    spec32 = pl.BlockSpec((B2, 32), lambda i: (i, 0))
    in_specs = [spec128, spec128, spec32]
    ops = [src, dst, he]
    in_specs += [_full(w) for w in weights]
    ops += list(weights)
    if last:
        out_shape = jax.ShapeDtypeStruct((EP2, 2), jnp.float32)
        out_specs = pl.BlockSpec((B2, 2), lambda i: (i, 0))
    else:
        out_shape = (jax.ShapeDtypeStruct((EP2, 32), jnp.float32),
                     jax.ShapeDtypeStruct((EP2, 128), jnp.float32))
        out_specs = (spec32, spec128)
    return pl.pallas_call(
        functools.partial(_edge_body, enc, last),
        grid=(NEB,),
        in_specs=in_specs,
        out_specs=out_specs,
        out_shape=out_shape,
    )(*ops)


def _encn_body(x_ref, W0, b0, W1, b1, out_ref):
    h = _relu(x_ref[...] @ W0[...] + b0[...])
    h = _relu(h @ W1[...] + b1[...])
    ids = lax.broadcasted_iota(jnp.int32, (N2, 1), 0).astype(jnp.float32)
    out_ref[...] = jnp.concatenate(
        [h, ids, jnp.zeros((N2, 31), jnp.float32)], axis=1)


def _node_body(p_ref, W0, b0, out_ref):
    a = p_ref[0] + p_ref[1]
    h = _relu(a @ W0[...] + b0[...])
    ids = lax.broadcasted_iota(jnp.int32, (N2, 1), 0).astype(jnp.float32)
    out_ref[...] = jnp.concatenate(
        [h, ids, jnp.zeros((N2, 31), jnp.float32)], axis=1)


def kernel(x, edge_index, edge_attr,
           enc_n_W0, enc_n_b0, enc_n_W1, enc_n_b1,
           enc_e_W0, enc_e_b0, enc_e_W1, enc_e_b1,
           em_W0, em_b0, em_W1, em_b1,
           fo_W0, fo_b0, fo_W1, fo_b1,
           fi_W0, fi_b0, fi_W1, fi_b1,
           nm_W0, nm_b0,
           cls_W0, cls_b0, cls_W1, cls_b1):
    f32 = jnp.float32
    ei = edge_index.reshape(2, NWORK, RPW, ROW_W)
    zero = jnp.zeros((NPT, 64), f32)

    def bd2(a):
        z = jnp.zeros_like(a)
        return jnp.concatenate(
            [jnp.concatenate([a, z], axis=1),
             jnp.concatenate([z, a], axis=1)], axis=0)

    def r2(b):
        return jnp.concatenate([b, b]).reshape(1, -1)

    def pad64(a):
        return jnp.concatenate([a, jnp.zeros_like(a)], axis=0)

    emS = bd2(pad64(em_W0[0:32]))
    emD = bd2(pad64(em_W0[32:64]))
    emE = bd2(em_W0[64:80])
    emW1 = bd2(em_W1)
    Wf0 = jnp.concatenate([fi_W0, fo_W0], axis=1)
    z56 = jnp.zeros((56, 32), f32)
    Wf1 = jnp.concatenate(
        [jnp.concatenate([fi_W1, z56], axis=1),
         jnp.concatenate([z56, fo_W1], axis=1)], axis=0)
    fS = bd2(pad64(Wf0[0:32]))
    fE = bd2(Wf0[32:48])
    fW1 = bd2(Wf1)
    fb0 = r2(jnp.concatenate([fi_b0, fo_b0]))
    fb1 = r2(jnp.concatenate([fi_b1, fo_b1]))

    r1 = lambda b: b.reshape(1, -1)
    x2 = jnp.concatenate([x, jnp.zeros((N2 - N, 128), f32)], axis=0)
    h_n = pl.pallas_call(
        _encn_body,
        out_shape=jax.ShapeDtypeStruct((N2, 64), f32),
    )(x2, enc_n_W0, r1(enc_n_b0), enc_n_W1, r1(enc_n_b1))

    gather_sc, scatter_sc = _sc_kernels()
    mid_w = (emS, emD, emE, r2(em_b0), emW1, r2(em_b1),
             fS, fE, fb0, fW1, fb1)
    he = edge_attr.reshape(EP2, 32)
    for step in range(3):
        src, dst = gather_sc(h_n, ei)
        src = src.reshape(EP2, 128)
        dst = dst.reshape(EP2, 128)
        if step == 0:
            weights = (bd2(enc_e_W0), r2(enc_e_b0),
                       bd2(enc_e_W1), r2(enc_e_b1)) + mid_w
        else:
            weights = mid_w
        he, ff = _edge_call(step == 0, False, src, dst, he, weights)
        partials = scatter_sc(ff.reshape(NWORK, RPW, ROW_W, 64), ei, zero)
        h_n = pl.pallas_call(
            _node_body,
            out_shape=jax.ShapeDtypeStruct((N2, 64), f32),
        )(partials, nm_W0, r1(nm_b0))

    src, dst = gather_sc(h_n, ei)
    src = src.reshape(EP2, 128)
    dst = dst.reshape(EP2, 128)
    weights = (emS, emD, emE, r2(em_b0), emW1, r2(em_b1),
               bd2(cls_W0), r2(cls_b0), bd2(cls_W1), r2(cls_b1))
    lg = _edge_call(False, True, src, dst, he, weights)
    return lg.reshape(E, 1)


# speedup vs baseline: 5.4444x; 1.0000x over previous
import functools

import jax
import jax.numpy as jnp
from jax import lax
from jax.experimental import pallas as pl
from jax.experimental.pallas import tpu as pltpu
from jax.experimental.pallas import tpu_sc as plsc

N = 10000
N2 = 10240
E = 320000
ROW_W = 80
NWORK = 32
RPW = E // ROW_W // NWORK
TPC = 16
NPT = N2 // TPC

EB = 16000
B2 = EB // 2
NEB = E // EB
EP2 = E // 2

GCH = 5
GNC = RPW // GCH
SCH = 5
SNC = RPW // SCH


@functools.lru_cache(maxsize=None)
def _sc_kernels():
    mesh = plsc.VectorSubcoreMesh(core_axis_name="c", subcore_axis_name="s",
                                  num_cores=2, num_subcores=16)

    @functools.partial(
        pl.kernel,
        out_type=(jax.ShapeDtypeStruct((NWORK, RPW, ROW_W, 64), jnp.float32),
                  jax.ShapeDtypeStruct((NWORK, RPW, ROW_W, 64), jnp.float32)),
        mesh=mesh,
        compiler_params=pltpu.CompilerParams(use_tc_tiling_on_sc=False),
        scratch_types=[
            pltpu.VMEM((RPW, ROW_W), jnp.int32),
            pltpu.VMEM((GCH, ROW_W, 64), jnp.float32),
            pltpu.SemaphoreType.DMA,
        ],
    )
    def _gather_sc(hn_hbm, ei_hbm, src_out, dst_out, idx_v, rows_v, sem):
        wid = lax.axis_index("s") * 2 + lax.axis_index("c")
        for which, out_hbm in ((0, src_out), (1, dst_out)):
            pltpu.sync_copy(ei_hbm.at[which, wid], idx_v)
            for i in range(GNC):
                cps = [pltpu.async_copy(hn_hbm.at[idx_v.at[i * GCH + j]],
                                        rows_v.at[j], sem)
                       for j in range(GCH)]
                for cp in cps:
                    cp.wait()
                pltpu.sync_copy(rows_v, out_hbm.at[wid, pl.ds(i * GCH, GCH)])

    @functools.partial(
        pl.kernel,
        out_type=jax.ShapeDtypeStruct((2, N2, 64), jnp.float32),
        mesh=mesh,
        compiler_params=pltpu.CompilerParams(use_tc_tiling_on_sc=False),
        scratch_types=[
            pltpu.VMEM((RPW, ROW_W), jnp.int32),
            pltpu.VMEM((SCH, ROW_W, 64), jnp.float32),
            pltpu.VMEM((NPT, 64), jnp.float32),
            pltpu.VMEM_SHARED((N2, 64), jnp.float32),
            pltpu.SemaphoreType.DMA,
        ],
    )
    def _scatter_sc(vals_hbm, ei_hbm, zero_hbm, out_hbm,
                    idx_v, vals_v, zb_v, acc_sh, sem):
        c = lax.axis_index("c")
        s = lax.axis_index("s")
        wid = s * 2 + c
        pltpu.sync_copy(zero_hbm, zb_v)
        pltpu.sync_copy(zb_v, acc_sh.at[pl.ds(s * NPT, NPT)])
        plsc.subcore_barrier()
        pltpu.sync_copy(ei_hbm.at[0, wid], idx_v)
        for i in range(SNC):
            pltpu.sync_copy(vals_hbm.at[wid, pl.ds(i * SCH, SCH)], vals_v)
            for j in range(SCH):
                pltpu.sync_copy(vals_v.at[j], acc_sh.at[idx_v.at[i * SCH + j]],
                                add=True)
        plsc.subcore_barrier()
        pltpu.sync_copy(acc_sh.at[pl.ds(s * NPT, NPT)],
                        out_hbm.at[c, pl.ds(s * NPT, NPT)])

    return _gather_sc, _scatter_sc


def _full(w):
    return pl.BlockSpec(w.shape, lambda i: (0,) * w.ndim)


def _relu(v):
    return jnp.maximum(v, 0.0)


def _edge_body(enc, last, *refs):
    it = iter(refs)
    src_ref, dst_ref, he_ref = next(it), next(it), next(it)
    if enc:
        eW0, eb0, eW1, eb1 = next(it), next(it), next(it), next(it)
    emS, emD, emE, emb0, emW1, emb1 = (next(it) for _ in range(6))
    if last:
        cW0, cb0, cW1, cb1 = next(it), next(it), next(it), next(it)
        lg_out = next(it)
    else:
        fS, fE, fb0, fW1, fb1 = (next(it) for _ in range(5))
        he_out, ff_out = next(it), next(it)

    src, dst, he = src_ref[...], dst_ref[...], he_ref[...]
    if enc:
        he = _relu(he @ eW0[...] + eb0[...])
        he = _relu(he @ eW1[...] + eb1[...])
    h1 = _relu(src @ emS[...] + dst @ emD[...] + he @ emE[...] + emb0[...])
    he2 = _relu(h1 @ emW1[...] + emb1[...])
    if last:
        c1 = _relu(he2 @ cW0[...] + cb0[...])
        lg_out[...] = c1 @ cW1[...] + cb1[...]
        return
    he_out[...] = he2
    g = _relu(src @ fS[...] + he2 @ fE[...] + fb0[...])
    ff = _relu(g @ fW1[...] + fb1[...])
    f32 = jnp.float32
    s0, s1 = src[:, 32:33], src[:, 96:97]
    d0, d1 = dst[:, 32:33], dst[:, 96:97]
    m = jnp.concatenate(
        [jnp.broadcast_to((s0 > d0).astype(f32), (B2, 32)),
         jnp.broadcast_to((s0 < d0).astype(f32), (B2, 32)),
         jnp.broadcast_to((s1 > d1).astype(f32), (B2, 32)),
         jnp.broadcast_to((s1 < d1).astype(f32), (B2, 32))], axis=1)
    ff_out[...] = ff * m


def _edge_call(enc, last, src, dst, he, weights):
    spec128 = pl.BlockSpec((B2, 128), lambda i: (i, 0))
    spec32 = pl.BlockSpec((B2, 32), lambda i: (i, 0))
    in_specs = [spec128, spec128, spec32]
    ops = [src, dst, he]
    in_specs += [_full(w) for w in weights]
    ops += list(weights)
    if last:
        out_shape = jax.ShapeDtypeStruct((EP2, 2), jnp.float32)
        out_specs = pl.BlockSpec((B2, 2), lambda i: (i, 0))
    else:
        out_shape = (jax.ShapeDtypeStruct((EP2, 32), jnp.float32),
                     jax.ShapeDtypeStruct((EP2, 128), jnp.float32))
        out_specs = (spec32, spec128)
    return pl.pallas_call(
        functools.partial(_edge_body, enc, last),
        grid=(NEB,),
        in_specs=in_specs,
        out_specs=out_specs,
        out_shape=out_shape,
    )(*ops)


def _encn_body(x_ref, W0, b0, W1, b1, out_ref):
    h = _relu(x_ref[...] @ W0[...] + b0[...])
    h = _relu(h @ W1[...] + b1[...])
    ids = lax.broadcasted_iota(jnp.int32, (N2, 1), 0).astype(jnp.float32)
    out_ref[...] = jnp.concatenate(
        [h, ids, jnp.zeros((N2, 31), jnp.float32)], axis=1)


def _node_body(p_ref, W0, b0, out_ref):
    a = p_ref[0] + p_ref[1]
    h = _relu(a @ W0[...] + b0[...])
    ids = lax.broadcasted_iota(jnp.int32, (N2, 1), 0).astype(jnp.float32)
    out_ref[...] = jnp.concatenate(
        [h, ids, jnp.zeros((N2, 31), jnp.float32)], axis=1)


def kernel(x, edge_index, edge_attr,
           enc_n_W0, enc_n_b0, enc_n_W1, enc_n_b1,
           enc_e_W0, enc_e_b0, enc_e_W1, enc_e_b1,
           em_W0, em_b0, em_W1, em_b1,
           fo_W0, fo_b0, fo_W1, fo_b1,
           fi_W0, fi_b0, fi_W1, fi_b1,
           nm_W0, nm_b0,
           cls_W0, cls_b0, cls_W1, cls_b1):
    f32 = jnp.float32
    ei = edge_index.reshape(2, NWORK, RPW, ROW_W)
    zero = jnp.zeros((NPT, 64), f32)

    def bd2(a):
        z = jnp.zeros_like(a)
        return jnp.concatenate(
            [jnp.concatenate([a, z], axis=1),
             jnp.concatenate([z, a], axis=1)], axis=0)

    def r2(b):
        return jnp.concatenate([b, b]).reshape(1, -1)

    def pad64(a):
        return jnp.concatenate([a, jnp.zeros_like(a)], axis=0)

    emS = bd2(pad64(em_W0[0:32]))
    emD = bd2(pad64(em_W0[32:64]))
    emE = bd2(em_W0[64:80])
    emW1 = bd2(em_W1)
    Wf0 = jnp.concatenate([fi_W0, fo_W0], axis=1)
    z56 = jnp.zeros((56, 32), f32)
    Wf1 = jnp.concatenate(
        [jnp.concatenate([fi_W1, z56], axis=1),
         jnp.concatenate([z56, fo_W1], axis=1)], axis=0)
    fS = bd2(pad64(Wf0[0:32]))
    fE = bd2(Wf0[32:48])
    fW1 = bd2(Wf1)
    fb0 = r2(jnp.concatenate([fi_b0, fo_b0]))
    fb1 = r2(jnp.concatenate([fi_b1, fo_b1]))

    r1 = lambda b: b.reshape(1, -1)
    x2 = jnp.concatenate([x, jnp.zeros((N2 - N, 128), f32)], axis=0)
    h_n = pl.pallas_call(
        _encn_body,
        out_shape=jax.ShapeDtypeStruct((N2, 64), f32),
    )(x2, enc_n_W0, r1(enc_n_b0), enc_n_W1, r1(enc_n_b1))

    gather_sc, scatter_sc = _sc_kernels()
    mid_w = (emS, emD, emE, r2(em_b0), emW1, r2(em_b1),
             fS, fE, fb0, fW1, fb1)
    he = edge_attr.reshape(EP2, 32)
    for step in range(3):
        src, dst = gather_sc(h_n, ei)
        src = src.reshape(EP2, 128)
        dst = dst.reshape(EP2, 128)
        if step == 0:
            weights = (bd2(enc_e_W0), r2(enc_e_b0),
                       bd2(enc_e_W1), r2(enc_e_b1)) + mid_w
        else:
            weights = mid_w
        he, ff = _edge_call(step == 0, False, src, dst, he, weights)
        partials = scatter_sc(ff.reshape(NWORK, RPW, ROW_W, 64), ei, zero)
        h_n = pl.pallas_call(
            _node_body,
            out_shape=jax.ShapeDtypeStruct((N2, 64), f32),
        )(partials, nm_W0, r1(nm_b0))

    src, dst = gather_sc(h_n, ei)
    src = src.reshape(EP2, 128)
    dst = dst.reshape(EP2, 128)
    weights = (emS, emD, emE, r2(em_b0), emW1, r2(em_b1),
               bd2(cls_W0), r2(cls_b0), bd2(cls_W1), r2(cls_b1))
    lg = _edge_call(False, True, src, dst, he, weights)
    return lg.reshape(E, 1)
